# baseline (speedup 1.0000x reference)
import functools

import jax
import jax.numpy as jnp
from jax import lax
from jax.experimental import pallas as pl
from jax.experimental.pallas import tpu as pltpu
from jax.experimental.pallas import tpu_sc as plsc

_ROWS, _COLS = 16384, 1024
_NW = 32
_ROWS_PER_W = _ROWS // _NW


def _sc_copy_body(x_hbm, out_hbm):
    wid = lax.axis_index("s") * 2 + lax.axis_index("c")
    base = wid * _ROWS_PER_W
    pltpu.sync_copy(
        x_hbm.at[pl.ds(base, _ROWS_PER_W), :],
        out_hbm.at[pl.ds(base, _ROWS_PER_W), :],
    )


@functools.partial(jax.jit, static_argnums=())
def kernel(x):
    mesh = plsc.VectorSubcoreMesh(core_axis_name="c", subcore_axis_name="s")
    fn = pl.kernel(
        _sc_copy_body,
        out_type=jax.ShapeDtypeStruct((_ROWS, _COLS), jnp.float32),
        mesh=mesh,
    )
    return fn(x)


# speedup vs baseline: 30.0934x; 30.0934x over previous
import functools

import jax
import jax.numpy as jnp
from jax import lax
from jax.experimental import pallas as pl
from jax.experimental.pallas import tpu as pltpu
from jax.experimental.pallas import tpu_sc as plsc

_ROWS, _COLS = 16384, 1024
_NW = 32
_ROWS_PER_W = _ROWS // _NW
_CHUNK = 32
_ITERS = _ROWS_PER_W // _CHUNK


def _sc_copy_body(x_hbm, out_hbm, buf0, buf1, isem0, isem1, osem0, osem1):
    wid = lax.axis_index("s") * 2 + lax.axis_index("c")
    base = wid * _ROWS_PER_W
    bufs = (buf0, buf1)
    isems = (isem0, isem1)
    osems = (osem0, osem1)

    def in_copy(i):
        b = i % 2
        return pltpu.make_async_copy(
            x_hbm.at[pl.ds(base + i * _CHUNK, _CHUNK), :], bufs[b], isems[b]
        )

    def out_copy(i):
        b = i % 2
        return pltpu.make_async_copy(
            bufs[b], out_hbm.at[pl.ds(base + i * _CHUNK, _CHUNK), :], osems[b]
        )

    in_copies = {0: in_copy(0)}
    in_copies[0].start()
    out_copies = {}
    for i in range(_ITERS):
        in_copies[i].wait()
        if i + 1 < _ITERS:
            if i - 1 >= 0:
                out_copies[i - 1].wait()
            in_copies[i + 1] = in_copy(i + 1)
            in_copies[i + 1].start()
        out_copies[i] = out_copy(i)
        out_copies[i].start()
    out_copies[_ITERS - 2].wait()
    out_copies[_ITERS - 1].wait()


def kernel(x):
    mesh = plsc.VectorSubcoreMesh(core_axis_name="c", subcore_axis_name="s")
    fn = pl.kernel(
        _sc_copy_body,
        out_type=jax.ShapeDtypeStruct((_ROWS, _COLS), jnp.float32),
        mesh=mesh,
        scratch_types=[
            pltpu.VMEM((_CHUNK, _COLS), jnp.float32),
            pltpu.VMEM((_CHUNK, _COLS), jnp.float32),
            pltpu.SemaphoreType.DMA,
            pltpu.SemaphoreType.DMA,
            pltpu.SemaphoreType.DMA,
            pltpu.SemaphoreType.DMA,
        ],
    )
    return fn(x)


# speedup vs baseline: 49.4266x; 1.6424x over previous
import jax
import jax.numpy as jnp
from jax.experimental import pallas as pl

_BLOCK_ROWS = 2048


def _copy_body(x_ref, o_ref):
    o_ref[...] = x_ref[...]


def kernel(x):
    rows, cols = x.shape
    grid = (rows // _BLOCK_ROWS,)
    return pl.pallas_call(
        _copy_body,
        grid=grid,
        in_specs=[pl.BlockSpec((_BLOCK_ROWS, cols), lambda i: (i, 0))],
        out_specs=pl.BlockSpec((_BLOCK_ROWS, cols), lambda i: (i, 0)),
        out_shape=jax.ShapeDtypeStruct((rows, cols), x.dtype),
    )(x)
